# Initial kernel scaffold; baseline (speedup 1.0000x reference)
import jax
import jax.numpy as jnp
from jax.experimental import pallas as pl


def kernel(x, seg, tok_table, pos_table, seg_table, gamma, beta):
    raise NotImplementedError("write your pallas kernel here")


# speedup vs baseline: 4.0840x; 4.0840x over previous
import functools

import jax
import jax.numpy as jnp
from jax import lax
from jax.experimental import pallas as pl
from jax.experimental.pallas import tpu as pltpu
from jax.experimental.pallas import tpu_sc as plsc

_NC = 2
_NS = 16
_NW = _NC * _NS
_L = 16

_D = 64
_BLK = 512
_IDXCHUNK = 128


def _rsqrt(tv):
    i = lax.bitcast_convert_type(tv, jnp.int32)
    i = jnp.int32(0x5F3759DF) - lax.shift_right_logical(i, 1)
    y = lax.bitcast_convert_type(i, jnp.float32)
    half = tv * 0.5
    for _ in range(3):
        y = y * (1.5 - half * y * y)
    return y


def _body(n_per_w, xf, segf, tok_tbl, ctbl, gamma, beta, out,
          xi, sg, ci, tb, cb, gv, bv, sem):
    wid = lax.axis_index("s") * _NC + lax.axis_index("c")
    base0 = wid * n_per_w
    nblk = n_per_w // _BLK

    pltpu.sync_copy(gamma, gv)
    pltpu.sync_copy(beta, bv)
    g = [gv[pl.ds(k * _L, _L)] for k in range(4)]
    b = [bv[pl.ds(k * _L, _L)] for k in range(4)]
    iota = lax.iota(jnp.int32, _L)

    def block(blk, carry):
        base = pl.multiple_of(base0 + blk * _BLK, _BLK)
        pltpu.sync_copy(xf.at[pl.ds(base, _BLK)], xi)
        pltpu.sync_copy(segf.at[pl.ds(base, _BLK)], sg)

        for gidx in range(_BLK // _L):
            posv = lax.rem(base + gidx * _L + iota, jnp.int32(200))
            ci[pl.ds(gidx * _L, _L)] = posv * 2 + sg[pl.ds(gidx * _L, _L)]

        copies = []
        for j in range(_BLK // _IDXCHUNK):
            sl = pl.ds(j * _IDXCHUNK, _IDXCHUNK)
            copies.append(pltpu.make_async_copy(
                tok_tbl.at[xi.at[sl]], tb.at[sl], sem))
            copies.append(pltpu.make_async_copy(
                ctbl.at[ci.at[sl]], cb.at[sl], sem))
        for c in copies:
            c.start()
        for c in copies:
            c.wait()

        def token(i, carry2):
            h = [tb[i, pl.ds(k * _L, _L)] + cb[i, pl.ds(k * _L, _L)]
                 for k in range(4)]
            s = (h[0] + h[1]) + (h[2] + h[3])
            q = (h[0] * h[0] + h[1] * h[1]) + (h[2] * h[2] + h[3] * h[3])
            tot = jnp.sum(s)
            totq = jnp.sum(q)
            mean = tot * (1.0 / _D)
            var = totq * (1.0 / _D) - mean * mean
            tv = jnp.full((_L,), var + 1e-5, jnp.float32)
            inv = _rsqrt(tv)
            for k in range(4):
                tb[i, pl.ds(k * _L, _L)] = (h[k] - mean) * (inv * g[k]) + b[k]
            return carry2

        lax.fori_loop(0, _BLK, token, 0)
        pltpu.sync_copy(tb, out.at[pl.ds(base, _BLK)])
        return carry

    lax.fori_loop(0, nblk, block, 0)


def kernel(x, seg, tok_table, pos_table, seg_table, gamma, beta):
    B, S = x.shape
    D = tok_table.shape[1]
    N = B * S
    n_per_w = N // _NW

    xf = x.reshape(N).astype(jnp.int32)
    segf = seg.reshape(N).astype(jnp.int32)
    ctbl = (pos_table[:S, None, :] + seg_table[None, :, :]).reshape(S * 2, D)

    mesh = plsc.VectorSubcoreMesh(core_axis_name="c", subcore_axis_name="s")
    run = pl.kernel(
        functools.partial(_body, n_per_w),
        out_type=jax.ShapeDtypeStruct((N, D), jnp.float32),
        mesh=mesh,
        compiler_params=pltpu.CompilerParams(
            needs_layout_passes=False, use_tc_tiling_on_sc=False),
        scratch_types=[
            pltpu.VMEM((_BLK,), jnp.int32),
            pltpu.VMEM((_BLK,), jnp.int32),
            pltpu.VMEM((_BLK,), jnp.int32),
            pltpu.VMEM((_BLK, _D), jnp.float32),
            pltpu.VMEM((_BLK, _D), jnp.float32),
            pltpu.VMEM((_D,), jnp.float32),
            pltpu.VMEM((_D,), jnp.float32),
            pltpu.SemaphoreType.DMA,
        ],
    )
    out = run(xf, segf, tok_table, ctbl, gamma, beta)
    return out.reshape(B, S, D)


# speedup vs baseline: 6.4326x; 1.5751x over previous
import functools

import jax
import jax.numpy as jnp
from jax import lax
from jax.experimental import pallas as pl
from jax.experimental.pallas import tpu as pltpu
from jax.experimental.pallas import tpu_sc as plsc

_NC = 2
_NS = 16
_NW = _NC * _NS
_L = 16

_D = 64
_BLK = 256
_CH = 128
_U = 4


def _rsqrt(tv):
    i = lax.bitcast_convert_type(tv, jnp.int32)
    i = jnp.int32(0x5F3759DF) - lax.shift_right_logical(i, 1)
    y = lax.bitcast_convert_type(i, jnp.float32)
    half = tv * 0.5
    for _ in range(2):
        y = y * (1.5 - half * y * y)
    return y


def _body(n_per_w, xf, cidxf, tok_tbl, ctbl, gamma, beta, out,
          xi0, xi1, ci0, ci1, tb0, tb1, cb0, cb1, gv, bv, st,
          sg0, sg1, sw0, sw1):
    xi = (xi0, xi1)
    ci = (ci0, ci1)
    tb = (tb0, tb1)
    cb = (cb0, cb1)
    sg = (sg0, sg1)
    sw = (sw0, sw1)
    wid = lax.axis_index("s") * _NC + lax.axis_index("c")
    base0 = wid * n_per_w
    nblk = n_per_w // _BLK

    pltpu.sync_copy(gamma, gv)
    pltpu.sync_copy(beta, bv)
    g = [gv[pl.ds(k * _L, _L)] for k in range(4)]
    bt = [bv[pl.ds(k * _L, _L)] for k in range(4)]

    def rowbase(blk):
        return pl.multiple_of(base0 + blk * _BLK, _BLK)

    def stage(blk, p):
        base = rowbase(blk)
        pltpu.sync_copy(xf.at[pl.ds(base, _BLK)], xi[p])
        pltpu.sync_copy(cidxf.at[pl.ds(base, _BLK)], ci[p])

    def gather_copies(p):
        cps = []
        for j in range(_BLK // _CH):
            sl = pl.ds(j * _CH, _CH)
            cps.append(pltpu.make_async_copy(
                tok_tbl.at[xi[p].at[sl]], tb[p].at[sl], sg[p]))
            cps.append(pltpu.make_async_copy(
                ctbl.at[ci[p].at[sl]], cb[p].at[sl], sg[p]))
        return cps

    def fire_gather(p):
        for c in gather_copies(p):
            c.start()

    def wait_gather(p):
        for c in gather_copies(p):
            c.wait()

    def wb_copy(blk, p):
        return pltpu.make_async_copy(tb[p], out.at[pl.ds(rowbase(blk), _BLK)],
                                     sw[p])

    def bcast(slot):
        return plsc.load_gather(st, [jnp.full((_L,), slot, jnp.int32)])

    def compute(p):
        tbp = tb[p]
        cbp = cb[p]

        def tok(i2, carry):
            i0 = i2 * _U
            for u in range(_U):
                i = i0 + u
                h = [tbp[i, pl.ds(k * _L, _L)] + cbp[i, pl.ds(k * _L, _L)]
                     for k in range(4)]
                s = (h[0] + h[1]) + (h[2] + h[3])
                q = (h[0] * h[0] + h[1] * h[1]) + (h[2] * h[2] + h[3] * h[3])
                st[pl.ds(u * 32, _L)] = plsc.cumsum(s * (1.0 / _D))
                st[pl.ds(u * 32 + _L, _L)] = plsc.cumsum(q * (1.0 / _D))
                mean = bcast(u * 32 + 15)
                eq = bcast(u * 32 + 31)
                inv = _rsqrt(eq - mean * mean + 1e-5)
                for k in range(4):
                    tbp[i, pl.ds(k * _L, _L)] = \
                        (h[k] - mean) * (inv * g[k]) + bt[k]
            return carry

        lax.fori_loop(0, _BLK // _U, tok, 0)

    stage(0, 0)
    fire_gather(0)
    stage(1, 1)
    fire_gather(1)
    wait_gather(0)
    compute(0)
    wb_copy(0, 0).start()

    def step(i2, carry):
        for off in (0, 1):
            b = 1 + 2 * i2 + off
            p = (1 + off) % 2
            pp = 1 - p
            stage(b + 1, pp)
            wb_copy(b - 1, pp).wait()
            fire_gather(pp)
            wait_gather(p)
            compute(p)
            wb_copy(b, p).start()
        return carry

    lax.fori_loop(0, (nblk - 2) // 2, step, 0)

    wb_copy(nblk - 2, 0).wait()
    wait_gather(1)
    compute(1)
    wb_copy(nblk - 1, 1).start()
    wb_copy(nblk - 1, 1).wait()


def kernel(x, seg, tok_table, pos_table, seg_table, gamma, beta):
    B, S = x.shape
    D = tok_table.shape[1]
    N = B * S
    n_per_w = N // _NW

    xf = x.reshape(N).astype(jnp.int32)
    cidxf = (jnp.arange(S, dtype=jnp.int32)[None, :] * 2
             + seg.astype(jnp.int32)).reshape(N)
    ctbl = (pos_table[:S, None, :] + seg_table[None, :, :]).reshape(S * 2, D)

    mesh = plsc.VectorSubcoreMesh(core_axis_name="c", subcore_axis_name="s")
    run = pl.kernel(
        functools.partial(_body, n_per_w),
        out_type=jax.ShapeDtypeStruct((N, D), jnp.float32),
        mesh=mesh,
        compiler_params=pltpu.CompilerParams(
            needs_layout_passes=False, use_tc_tiling_on_sc=False),
        scratch_types=[
            pltpu.VMEM((_BLK,), jnp.int32),
            pltpu.VMEM((_BLK,), jnp.int32),
            pltpu.VMEM((_BLK,), jnp.int32),
            pltpu.VMEM((_BLK,), jnp.int32),
            pltpu.VMEM((_BLK, _D), jnp.float32),
            pltpu.VMEM((_BLK, _D), jnp.float32),
            pltpu.VMEM((_BLK, _D), jnp.float32),
            pltpu.VMEM((_BLK, _D), jnp.float32),
            pltpu.VMEM((_D,), jnp.float32),
            pltpu.VMEM((_D,), jnp.float32),
            pltpu.VMEM((_U * 32,), jnp.float32),
            pltpu.SemaphoreType.DMA,
            pltpu.SemaphoreType.DMA,
            pltpu.SemaphoreType.DMA,
            pltpu.SemaphoreType.DMA,
        ],
    )
    out = run(xf, cidxf, tok_table, ctbl, gamma, beta)
    return out.reshape(B, S, D)
